# Initial kernel scaffold; baseline (speedup 1.0000x reference)
import jax
import jax.numpy as jnp
from jax.experimental import pallas as pl


def kernel(x, edge_index, W0_nbr, W0_root, b0, W1_nbr, W1_root, b1, W2_nbr, W2_root, b2):
    raise NotImplementedError("write your pallas kernel here")


# speedup vs baseline: 3.1087x; 3.1087x over previous
import functools

import jax
import jax.numpy as jnp
from jax import lax
from jax.experimental import pallas as pl
from jax.experimental.pallas import tpu as pltpu
from jax.experimental.pallas import tpu_sc as plsc

N = 10000
E = 320000
D = 128
NC = 2
NS = 16
NW = NC * NS
NPAD = 10240
TRASH = N
CHUNK = 80
EDGES_PER_TILE = E // NW
PCHUNKS = 128
PAD_EDGES = PCHUNKS * CHUNK - EDGES_PER_TILE
ROWS_PER_TILE = NPAD // NS
ZBLK = 16
PB = 2000

_sc_mesh = plsc.VectorSubcoreMesh(core_axis_name="c", subcore_axis_name="s")


def _remap_body(src_hbm, dst_hbm, out_hbm, sbuf, dbuf):
    c = lax.axis_index("c")
    s = lax.axis_index("s")
    wid = c * NS + s
    base = wid * EDGES_PER_TILE

    def chunk(k, carry):
        off = base + k * PB
        pltpu.sync_copy(src_hbm.at[pl.ds(off, PB)], sbuf)
        pltpu.sync_copy(dst_hbm.at[pl.ds(off, PB)], dbuf)

        def inner(i, carry2):
            sv = sbuf[pl.ds(i * 16, 16)]
            dv = dbuf[pl.ds(i * 16, 16)]
            dbuf[pl.ds(i * 16, 16)] = jnp.where(sv == dv, TRASH, dv)
            return carry2

        lax.fori_loop(0, PB // 16, inner, 0)
        pltpu.sync_copy(dbuf, out_hbm.at[pl.ds(off, PB)])
        return carry

    lax.fori_loop(0, EDGES_PER_TILE // PB, chunk, 0)


_remap = pl.kernel(
    _remap_body,
    out_type=jax.ShapeDtypeStruct((E,), jnp.int32),
    mesh=_sc_mesh,
    scratch_types=[
        pltpu.VMEM((PB,), jnp.int32),
        pltpu.VMEM((PB,), jnp.int32),
    ],
)


def _segsum_body(h_hbm, src_hbm, dstr_hbm, zeros_hbm, out_hbm,
                 src_t, dst_t, rows, zstage, acc, sem):
    c = lax.axis_index("c")
    s = lax.axis_index("s")
    wid = c * NS + s

    pltpu.sync_copy(zeros_hbm, zstage)
    for t in range(ROWS_PER_TILE // ZBLK):
        pltpu.sync_copy(zstage, acc.at[pl.ds(s * ROWS_PER_TILE + t * ZBLK, ZBLK)])

    pltpu.sync_copy(src_hbm.at[wid], src_t)
    pltpu.sync_copy(dstr_hbm.at[wid], dst_t)
    plsc.subcore_barrier()

    def chunk(j, carry):
        pltpu.async_copy(h_hbm.at[src_t.at[j]], rows, sem).wait()
        pltpu.sync_copy(rows, acc.at[dst_t.at[j]], add=True)
        return carry

    lax.fori_loop(0, PCHUNKS, chunk, 0)
    plsc.subcore_barrier()

    rbase = s * ROWS_PER_TILE
    pltpu.sync_copy(acc.at[pl.ds(rbase, ROWS_PER_TILE)],
                    out_hbm.at[c, pl.ds(rbase, ROWS_PER_TILE)])


_segsum = pl.kernel(
    _segsum_body,
    out_type=jax.ShapeDtypeStruct((NC, NPAD, D), jnp.float32),
    mesh=_sc_mesh,
    scratch_types=[
        pltpu.VMEM((PCHUNKS, CHUNK), jnp.int32),
        pltpu.VMEM((PCHUNKS, CHUNK), jnp.int32),
        pltpu.VMEM((CHUNK, D), jnp.float32),
        pltpu.VMEM((ZBLK, D), jnp.float32),
        pltpu.VMEM_SHARED((NPAD, D), jnp.float32),
        pltpu.SemaphoreType.DMA,
    ],
)


RB = 400
_GRID = N // RB


def _dense_body(relu, h_ref, s_ref, wn_ref, wr_ref, b_ref, o_ref):
    hb = h_ref[...]
    sb = s_ref[0] + s_ref[1]
    wn = wn_ref[...]
    wc = wn + wr_ref[...]
    y = jnp.dot(sb, wn, preferred_element_type=jnp.float32)
    y = y + jnp.dot(hb, wc, preferred_element_type=jnp.float32)
    y = y + b_ref[...]
    if relu:
        y = jnp.maximum(y, 0.0)
    o_ref[...] = hb + y


def _dense(h, s2, wn, wr, b, relu):
    return pl.pallas_call(
        functools.partial(_dense_body, relu),
        grid=(_GRID,),
        in_specs=[
            pl.BlockSpec((RB, D), lambda i: (i, 0)),
            pl.BlockSpec((NC, RB, D), lambda i: (0, i, 0)),
            pl.BlockSpec((D, D), lambda i: (0, 0)),
            pl.BlockSpec((D, D), lambda i: (0, 0)),
            pl.BlockSpec((1, D), lambda i: (0, 0)),
        ],
        out_specs=pl.BlockSpec((RB, D), lambda i: (i, 0)),
        out_shape=jax.ShapeDtypeStruct((N, D), jnp.float32),
    )(h, s2, wn, wr, b)


def kernel(x, edge_index,
           W0_nbr, W0_root, b0,
           W1_nbr, W1_root, b1,
           W2_nbr, W2_root, b2):
    src = edge_index[0]
    dst = edge_index[1]
    dstr = _remap(src, dst)
    spad = jnp.zeros((NW, PAD_EDGES), jnp.int32)
    dpad = jnp.full((NW, PAD_EDGES), TRASH, jnp.int32)
    src_t = jnp.concatenate([src.reshape(NW, EDGES_PER_TILE), spad],
                            axis=1).reshape(NW, PCHUNKS, CHUNK)
    dstr_t = jnp.concatenate([dstr.reshape(NW, EDGES_PER_TILE), dpad],
                             axis=1).reshape(NW, PCHUNKS, CHUNK)
    zeros = jnp.zeros((ZBLK, D), jnp.float32)

    h = x
    for wn, wr, b, relu in ((W0_nbr, W0_root, b0, True),
                            (W1_nbr, W1_root, b1, True),
                            (W2_nbr, W2_root, b2, False)):
        s2 = _segsum(h, src_t, dstr_t, zeros)
        h = _dense(h, s2, wn, wr, b.reshape(1, D), relu)
    return h


# speedup vs baseline: 3.7754x; 1.2145x over previous
import functools

import jax
import jax.numpy as jnp
from jax import lax
from jax.experimental import pallas as pl
from jax.experimental.pallas import tpu as pltpu
from jax.experimental.pallas import tpu_sc as plsc

N = 10000
E = 320000
D = 128
NC = 2
NS = 16
NW = NC * NS
NPAD = 10240
TRASH = N
CHUNK = 128
EDGES_PER_TILE = E // NW
PCHUNKS = 80
HALF = PCHUNKS // 2
PAD_EDGES = PCHUNKS * CHUNK - EDGES_PER_TILE
ROWS_PER_TILE = NPAD // NS
PB = 2000

_sc_mesh = plsc.VectorSubcoreMesh(core_axis_name="c", subcore_axis_name="s")


def _remap_body(src_hbm, dst_hbm, out_hbm, sbuf, dbuf):
    c = lax.axis_index("c")
    s = lax.axis_index("s")
    wid = c * NS + s
    base = wid * EDGES_PER_TILE

    def chunk(k, carry):
        off = base + k * PB
        pltpu.sync_copy(src_hbm.at[pl.ds(off, PB)], sbuf)
        pltpu.sync_copy(dst_hbm.at[pl.ds(off, PB)], dbuf)

        def inner(i, carry2):
            sv = sbuf[pl.ds(i * 16, 16)]
            dv = dbuf[pl.ds(i * 16, 16)]
            dbuf[pl.ds(i * 16, 16)] = jnp.where(sv == dv, TRASH, dv)
            return carry2

        lax.fori_loop(0, PB // 16, inner, 0)
        pltpu.sync_copy(dbuf, out_hbm.at[pl.ds(off, PB)])
        return carry

    lax.fori_loop(0, EDGES_PER_TILE // PB, chunk, 0)


_remap = pl.kernel(
    _remap_body,
    out_type=jax.ShapeDtypeStruct((E,), jnp.int32),
    mesh=_sc_mesh,
    scratch_types=[
        pltpu.VMEM((PB,), jnp.int32),
        pltpu.VMEM((PB,), jnp.int32),
    ],
)


def _segsum_body(h_hbm, idx_hbm, zeros_hbm, out_hbm,
                 idx_t, row0, row1, acc, sem0, sem1):
    c = lax.axis_index("c")
    s = lax.axis_index("s")
    wid = c * NS + s

    pltpu.sync_copy(zeros_hbm, row0)
    for t in range(ROWS_PER_TILE // CHUNK):
        pltpu.sync_copy(row0, acc.at[pl.ds(s * ROWS_PER_TILE + t * CHUNK, CHUNK)])
    plsc.subcore_barrier()

    for hh in range(PCHUNKS // HALF):
        pltpu.sync_copy(idx_hbm.at[wid, pl.ds(hh * HALF, HALF)], idx_t)
        pltpu.async_copy(h_hbm.at[idx_t.at[0, 0]], row0, sem0)

        def pair(kk, carry):
            j0 = 2 * kk
            j1 = j0 + 1
            pltpu.async_copy(h_hbm.at[idx_t.at[j1, 0]], row1, sem1)
            pltpu.make_async_copy(h_hbm.at[idx_t.at[j0, 0]], row0, sem0).wait()
            pltpu.sync_copy(row0, acc.at[idx_t.at[j0, 1]], add=True)
            @pl.when(kk < HALF // 2 - 1)
            def _():
                pltpu.async_copy(h_hbm.at[idx_t.at[j0 + 2, 0]], row0, sem0)
            pltpu.make_async_copy(h_hbm.at[idx_t.at[j1, 0]], row1, sem1).wait()
            pltpu.sync_copy(row1, acc.at[idx_t.at[j1, 1]], add=True)
            return carry

        lax.fori_loop(0, HALF // 2, pair, 0)

    plsc.subcore_barrier()
    rbase = s * ROWS_PER_TILE
    pltpu.sync_copy(acc.at[pl.ds(rbase, ROWS_PER_TILE)],
                    out_hbm.at[c, pl.ds(rbase, ROWS_PER_TILE)])


_segsum = pl.kernel(
    _segsum_body,
    out_type=jax.ShapeDtypeStruct((NC, NPAD, D), jnp.float32),
    mesh=_sc_mesh,
    scratch_types=[
        pltpu.VMEM((HALF, 2, CHUNK), jnp.int32),
        pltpu.VMEM((CHUNK, D), jnp.float32),
        pltpu.VMEM((CHUNK, D), jnp.float32),
        pltpu.VMEM_SHARED((NPAD, D), jnp.float32),
        pltpu.SemaphoreType.DMA,
        pltpu.SemaphoreType.DMA,
    ],
)


RB = 400
_GRID = N // RB


def _dense_body(relu, h_ref, s_ref, wn_ref, wr_ref, b_ref, o_ref):
    hb = h_ref[...]
    sb = s_ref[0] + s_ref[1]
    wn = wn_ref[...]
    wc = wn + wr_ref[...]
    y = jnp.dot(sb, wn, preferred_element_type=jnp.float32)
    y = y + jnp.dot(hb, wc, preferred_element_type=jnp.float32)
    y = y + b_ref[...]
    if relu:
        y = jnp.maximum(y, 0.0)
    o_ref[...] = hb + y


def _dense(h, s2, wn, wr, b, relu):
    return pl.pallas_call(
        functools.partial(_dense_body, relu),
        grid=(_GRID,),
        in_specs=[
            pl.BlockSpec((RB, D), lambda i: (i, 0)),
            pl.BlockSpec((NC, RB, D), lambda i: (0, i, 0)),
            pl.BlockSpec((D, D), lambda i: (0, 0)),
            pl.BlockSpec((D, D), lambda i: (0, 0)),
            pl.BlockSpec((1, D), lambda i: (0, 0)),
        ],
        out_specs=pl.BlockSpec((RB, D), lambda i: (i, 0)),
        out_shape=jax.ShapeDtypeStruct((N, D), jnp.float32),
    )(h, s2, wn, wr, b)


def kernel(x, edge_index,
           W0_nbr, W0_root, b0,
           W1_nbr, W1_root, b1,
           W2_nbr, W2_root, b2):
    src = edge_index[0]
    dst = edge_index[1]
    dstr = _remap(src, dst)
    spad = jnp.zeros((NW, PAD_EDGES), jnp.int32)
    dpad = jnp.full((NW, PAD_EDGES), TRASH, jnp.int32)
    src_t = jnp.concatenate([src.reshape(NW, EDGES_PER_TILE), spad],
                            axis=1).reshape(NW, PCHUNKS, 1, CHUNK)
    dstr_t = jnp.concatenate([dstr.reshape(NW, EDGES_PER_TILE), dpad],
                             axis=1).reshape(NW, PCHUNKS, 1, CHUNK)
    idx = jnp.concatenate([src_t, dstr_t], axis=2)
    zeros = jnp.zeros((CHUNK, D), jnp.float32)

    h = x
    for wn, wr, b, relu in ((W0_nbr, W0_root, b0, True),
                            (W1_nbr, W1_root, b1, True),
                            (W2_nbr, W2_root, b2, False)):
        s2 = _segsum(h, idx, zeros)
        h = _dense(h, s2, wn, wr, b.reshape(1, D), relu)
    return h


# speedup vs baseline: 11.4767x; 3.0398x over previous
import functools

import jax
import jax.numpy as jnp
from jax import lax
from jax.experimental import pallas as pl
from jax.experimental.pallas import tpu as pltpu
from jax.experimental.pallas import tpu_sc as plsc

N = 10000
E = 320000
D = 128
NC = 2
NS = 16
NW = NC * NS
NPAD = 10240
TRASH = N
CHUNK = 100
EDGES_PER_TILE = E // NW
PCHUNKS = 100
QTR = PCHUNKS // 4
ROWS_PER_TILE = NPAD // NS
PB = 2000

_sc_mesh = plsc.VectorSubcoreMesh(core_axis_name="c", subcore_axis_name="s")


def _remap_body(src_hbm, dst_hbm, out_hbm, sbuf, dbuf):
    c = lax.axis_index("c")
    s = lax.axis_index("s")
    wid = c * NS + s
    base = wid * EDGES_PER_TILE

    def chunk(k, carry):
        off = base + k * PB
        pltpu.sync_copy(src_hbm.at[pl.ds(off, PB)], sbuf)
        pltpu.sync_copy(dst_hbm.at[pl.ds(off, PB)], dbuf)

        def inner(i, carry2):
            sv = sbuf[pl.ds(i * 16, 16)]
            dv = dbuf[pl.ds(i * 16, 16)]
            dbuf[pl.ds(i * 16, 16)] = jnp.where(sv == dv, TRASH, dv)
            return carry2

        lax.fori_loop(0, PB // 16, inner, 0)
        pltpu.sync_copy(dbuf, out_hbm.at[pl.ds(off, PB)])
        return carry

    lax.fori_loop(0, EDGES_PER_TILE // PB, chunk, 0)


_remap = pl.kernel(
    _remap_body,
    out_type=jax.ShapeDtypeStruct((E,), jnp.int32),
    mesh=_sc_mesh,
    scratch_types=[
        pltpu.VMEM((PB,), jnp.int32),
        pltpu.VMEM((PB,), jnp.int32),
    ],
)


def _segsum_body(h_hbm, idx_hbm, zeros_hbm, out_hbm,
                 idx_t, row0, row1, row2, acc, sem0, sem1, sem2):
    c = lax.axis_index("c")
    s = lax.axis_index("s")
    wid = c * NS + s

    pltpu.sync_copy(zeros_hbm, row0)
    for t in range(ROWS_PER_TILE // CHUNK):
        pltpu.sync_copy(row0, acc.at[pl.ds(s * ROWS_PER_TILE + t * CHUNK, CHUNK)])
    rem = ROWS_PER_TILE % CHUNK
    if rem:
        pltpu.sync_copy(
            row0.at[pl.ds(0, rem)],
            acc.at[pl.ds(s * ROWS_PER_TILE + (ROWS_PER_TILE // CHUNK) * CHUNK,
                         rem)])
    plsc.subcore_barrier()

    rows = (row0, row1, row2)
    sems = (sem0, sem1, sem2)

    def gather(j, b):
        pltpu.async_copy(h_hbm.at[idx_t.at[j, 0]], rows[b], sems[b])

    def drain_scatter(j, b):
        pltpu.make_async_copy(h_hbm.at[idx_t.at[j, 0]], rows[b], sems[b]).wait()
        pltpu.sync_copy(rows[b], acc.at[idx_t.at[j, 1]], add=True)

    for hh in range(PCHUNKS // QTR):
        pltpu.sync_copy(idx_hbm.at[wid, pl.ds(hh * QTR, QTR)], idx_t)
        gather(0, 0)
        gather(1, 1)

        def ring(kk, carry):
            j = 3 * kk
            gather(j + 2, 2)
            drain_scatter(j, 0)
            gather(j + 3, 0)
            drain_scatter(j + 1, 1)
            @pl.when(kk < QTR // 3 - 1)
            def _():
                gather(j + 4, 1)
            drain_scatter(j + 2, 2)
            return carry

        lax.fori_loop(0, QTR // 3, ring, 0)
        drain_scatter(QTR - 1, 0)

    plsc.subcore_barrier()
    rbase = s * ROWS_PER_TILE
    pltpu.sync_copy(acc.at[pl.ds(rbase, ROWS_PER_TILE)],
                    out_hbm.at[c, pl.ds(rbase, ROWS_PER_TILE)])


_segsum = pl.kernel(
    _segsum_body,
    out_type=jax.ShapeDtypeStruct((NC, NPAD, D), jnp.float32),
    mesh=_sc_mesh,
    scratch_types=[
        pltpu.VMEM((QTR, 2, CHUNK), jnp.int32),
        pltpu.VMEM((CHUNK, D), jnp.float32),
        pltpu.VMEM((CHUNK, D), jnp.float32),
        pltpu.VMEM((CHUNK, D), jnp.float32),
        pltpu.VMEM_SHARED((NPAD, D), jnp.float32),
        pltpu.SemaphoreType.DMA,
        pltpu.SemaphoreType.DMA,
        pltpu.SemaphoreType.DMA,
    ],
)


RB = 400
_GRID = N // RB


def _dense_body(relu, h_ref, s_ref, wn_ref, wr_ref, b_ref, o_ref):
    hb = h_ref[...]
    sb = s_ref[0] + s_ref[1]
    wn = wn_ref[...]
    wc = wn + wr_ref[...]
    y = jnp.dot(sb, wn, preferred_element_type=jnp.float32)
    y = y + jnp.dot(hb, wc, preferred_element_type=jnp.float32)
    y = y + b_ref[...]
    if relu:
        y = jnp.maximum(y, 0.0)
    o_ref[...] = hb + y


def _dense(h, s2, wn, wr, b, relu):
    return pl.pallas_call(
        functools.partial(_dense_body, relu),
        grid=(_GRID,),
        in_specs=[
            pl.BlockSpec((RB, D), lambda i: (i, 0)),
            pl.BlockSpec((NC, RB, D), lambda i: (0, i, 0)),
            pl.BlockSpec((D, D), lambda i: (0, 0)),
            pl.BlockSpec((D, D), lambda i: (0, 0)),
            pl.BlockSpec((1, D), lambda i: (0, 0)),
        ],
        out_specs=pl.BlockSpec((RB, D), lambda i: (i, 0)),
        out_shape=jax.ShapeDtypeStruct((N, D), jnp.float32),
    )(h, s2, wn, wr, b)


def kernel(x, edge_index,
           W0_nbr, W0_root, b0,
           W1_nbr, W1_root, b1,
           W2_nbr, W2_root, b2):
    src = edge_index[0]
    dst = edge_index[1]
    dstr = _remap(src, dst)
    src_t = src.reshape(NW, PCHUNKS, 1, CHUNK)
    dstr_t = dstr.reshape(NW, PCHUNKS, 1, CHUNK)
    idx = jnp.concatenate([src_t, dstr_t], axis=2)
    zeros = jnp.zeros((CHUNK, D), jnp.float32)

    h = x
    for wn, wr, b, relu in ((W0_nbr, W0_root, b0, True),
                            (W1_nbr, W1_root, b1, True),
                            (W2_nbr, W2_root, b2, False)):
        s2 = _segsum(h, idx, zeros)
        h = _dense(h, s2, wn, wr, b.reshape(1, D), relu)
    return h
